# Initial kernel scaffold; baseline (speedup 1.0000x reference)
import functools

import jax
import jax.numpy as jnp
from jax import lax
from jax.experimental import pallas as pl
from jax.experimental.pallas import tpu as pltpu
from jax.experimental.pallas import tpu_sc as plsc

N_NODES = 10000
N_EDGES = 160000
IN_DIM = 256
HID_DIM = 512

HALF = IN_DIM // 2
WCOLS = HALF + 16
NC = 2
NS = 16
EDGES_PER_TILE = N_EDGES // NS
CHUNK = 200
NCHUNK = EDGES_PER_TILE // CHUNK
ROWS_PER_TILE = N_NODES // NS


def _sc_scatter(aug, src3, dst4, zrows, out, src_v, dst_v, rows_v, shared, sem):
    c = lax.axis_index("c")
    s = lax.axis_index("s")
    pltpu.sync_copy(zrows, shared.at[pl.ds(s * ROWS_PER_TILE, ROWS_PER_TILE)])
    pltpu.sync_copy(src3.at[s], src_v)
    pltpu.sync_copy(dst4.at[c, s], dst_v)
    plsc.subcore_barrier()

    def body(j, carry):
        pltpu.async_copy(aug.at[dst_v.at[j]], rows_v, sem).wait()
        pltpu.sync_copy(rows_v, shared.at[src_v.at[j]], add=True)
        return carry

    lax.fori_loop(0, NCHUNK, body, 0)
    plsc.subcore_barrier()
    pltpu.sync_copy(
        shared.at[pl.ds(s * ROWS_PER_TILE, ROWS_PER_TILE)],
        out.at[c, pl.ds(s * ROWS_PER_TILE, ROWS_PER_TILE)],
    )


_sc_scatter_call = functools.partial(
    pl.kernel,
    out_type=jax.ShapeDtypeStruct((NC, N_NODES, WCOLS), jnp.float32),
    mesh=plsc.VectorSubcoreMesh(core_axis_name="c", subcore_axis_name="s"),
    scratch_types=[
        pltpu.VMEM((NCHUNK, CHUNK), jnp.int32),
        pltpu.VMEM((NCHUNK, CHUNK), jnp.int32),
        pltpu.VMEM((CHUNK, WCOLS), jnp.float32),
        pltpu.VMEM_SHARED((N_NODES, WCOLS), jnp.float32),
        pltpu.SemaphoreType.DMA,
    ],
)(_sc_scatter)


def _tc_body(x_ref, n_ref, ws_ref, wn0_ref, wn1_ref, b_ref, o_ref):
    x = x_ref[...]
    nb = n_ref[...]
    sum0 = nb[0, :, :HALF]
    sum1 = nb[1, :, :HALF]
    cnt = nb[0, :, HALF:HALF + 1]
    mask = cnt > 0.0
    safe = jnp.where(mask, cnt, 1.0)
    m0 = jnp.where(mask, sum0 / safe, 0.0)
    m1 = jnp.where(mask, sum1 / safe, 0.0)
    acc = jnp.dot(x, ws_ref[...], preferred_element_type=jnp.float32,
                  precision=lax.Precision.HIGHEST)
    acc += jnp.dot(m0, wn0_ref[...], preferred_element_type=jnp.float32,
                   precision=lax.Precision.HIGHEST)
    acc += jnp.dot(m1, wn1_ref[...], preferred_element_type=jnp.float32,
                   precision=lax.Precision.HIGHEST)
    acc += b_ref[...]
    acc = jnp.maximum(acc, 0.0)
    nrm = jnp.sqrt(jnp.sum(acc * acc, axis=1, keepdims=True)) + 1e-9
    o_ref[...] = acc / nrm


def _tc_call(x, neigh, ws, wn0, wn1, b):
    R = 1000
    grid = (N_NODES // R,)
    return pl.pallas_call(
        _tc_body,
        grid=grid,
        in_specs=[
            pl.BlockSpec((R, IN_DIM), lambda i: (i, 0)),
            pl.BlockSpec((NC, R, WCOLS), lambda i: (0, i, 0)),
            pl.BlockSpec((IN_DIM, HID_DIM), lambda i: (0, 0)),
            pl.BlockSpec((HALF, HID_DIM), lambda i: (0, 0)),
            pl.BlockSpec((HALF, HID_DIM), lambda i: (0, 0)),
            pl.BlockSpec((1, HID_DIM), lambda i: (0, 0)),
        ],
        out_specs=pl.BlockSpec((R, HID_DIM), lambda i: (i, 0)),
        out_shape=jax.ShapeDtypeStruct((N_NODES, HID_DIM), jnp.float32),
    )(x, neigh, ws, wn0, wn1, b)


@jax.jit
def kernel(item_emb, edges, w_self_W, w_self_b, w_neigh_W, w_neigh_b):
    f32 = jnp.float32
    src = edges[:, 0].astype(jnp.int32)
    dst = edges[:, 1].astype(jnp.int32)
    ones = jnp.ones((N_NODES, 1), f32)
    pad = jnp.zeros((N_NODES, WCOLS - HALF - 1), f32)
    aug = jnp.concatenate([
        jnp.concatenate([item_emb[:, :HALF], ones, pad], axis=1),
        jnp.concatenate([item_emb[:, HALF:], ones, pad], axis=1),
    ], axis=0)
    src3 = src.reshape(NS, NCHUNK, CHUNK)
    dst4 = jnp.stack([dst, dst + N_NODES]).reshape(NC, NS, NCHUNK, CHUNK)
    zrows = jnp.zeros((ROWS_PER_TILE, WCOLS), f32)

    neigh = _sc_scatter_call(aug, src3, dst4, zrows)

    bias = (w_self_b + w_neigh_b).reshape(1, HID_DIM)
    return _tc_call(item_emb, neigh, w_self_W,
                    w_neigh_W[:HALF], w_neigh_W[HALF:], bias)


# speedup vs baseline: 3.8142x; 3.8142x over previous
import functools

import jax
import jax.numpy as jnp
from jax import lax
from jax.experimental import pallas as pl
from jax.experimental.pallas import tpu as pltpu
from jax.experimental.pallas import tpu_sc as plsc

N_NODES = 10000
N_EDGES = 160000
IN_DIM = 256
HID_DIM = 512

HALF = IN_DIM // 2
WCOLS = HALF + 16
NC = 2
NS = 16
EDGES_PER_TILE = N_EDGES // NS
CHUNK = 200
NCHUNK = EDGES_PER_TILE // CHUNK
ROWS_PER_TILE = N_NODES // NS


def _sc_scatter(aug, src3, dst4, zrows, out, src_v, dst_v, rows_v, shared, sem):
    c = lax.axis_index("c")
    s = lax.axis_index("s")
    pltpu.sync_copy(zrows, shared.at[pl.ds(s * ROWS_PER_TILE, ROWS_PER_TILE)])
    plsc.subcore_barrier()

    def body(j, carry):
        pltpu.sync_copy(src3.at[s, j], src_v)
        pltpu.sync_copy(dst4.at[c, s, j], dst_v)
        pltpu.async_copy(aug.at[dst_v], rows_v, sem).wait()
        pltpu.sync_copy(rows_v, shared.at[src_v], add=True)
        return carry

    lax.fori_loop(0, NCHUNK, body, 0)
    plsc.subcore_barrier()
    pltpu.sync_copy(
        shared.at[pl.ds(s * ROWS_PER_TILE, ROWS_PER_TILE)],
        out.at[c, pl.ds(s * ROWS_PER_TILE, ROWS_PER_TILE)],
    )


_sc_scatter_call = functools.partial(
    pl.kernel,
    out_type=jax.ShapeDtypeStruct((NC, N_NODES, WCOLS), jnp.float32),
    mesh=plsc.VectorSubcoreMesh(core_axis_name="c", subcore_axis_name="s"),
    scratch_types=[
        pltpu.VMEM((CHUNK,), jnp.int32),
        pltpu.VMEM((CHUNK,), jnp.int32),
        pltpu.VMEM((CHUNK, WCOLS), jnp.float32),
        pltpu.VMEM_SHARED((N_NODES, WCOLS), jnp.float32),
        pltpu.SemaphoreType.DMA,
    ],
    compiler_params=pltpu.CompilerParams(use_tc_tiling_on_sc=False),
)(_sc_scatter)


def _tc_body(x_ref, n_ref, ws_ref, wn0_ref, wn1_ref, b_ref, o_ref):
    x = x_ref[...]
    nb = n_ref[...]
    sum0 = nb[0, :, :HALF]
    sum1 = nb[1, :, :HALF]
    cnt = nb[0, :, HALF:HALF + 1]
    mask = cnt > 0.0
    safe = jnp.where(mask, cnt, 1.0)
    m0 = jnp.where(mask, sum0 / safe, 0.0)
    m1 = jnp.where(mask, sum1 / safe, 0.0)
    acc = jnp.dot(x, ws_ref[...], preferred_element_type=jnp.float32,
                  precision=lax.Precision.HIGHEST)
    acc += jnp.dot(m0, wn0_ref[...], preferred_element_type=jnp.float32,
                   precision=lax.Precision.HIGHEST)
    acc += jnp.dot(m1, wn1_ref[...], preferred_element_type=jnp.float32,
                   precision=lax.Precision.HIGHEST)
    acc += b_ref[...]
    acc = jnp.maximum(acc, 0.0)
    nrm = jnp.sqrt(jnp.sum(acc * acc, axis=1, keepdims=True)) + 1e-9
    o_ref[...] = acc / nrm


def _tc_call(x, neigh, ws, wn0, wn1, b):
    R = 1000
    grid = (N_NODES // R,)
    return pl.pallas_call(
        _tc_body,
        grid=grid,
        in_specs=[
            pl.BlockSpec((R, IN_DIM), lambda i: (i, 0)),
            pl.BlockSpec((NC, R, WCOLS), lambda i: (0, i, 0)),
            pl.BlockSpec((IN_DIM, HID_DIM), lambda i: (0, 0)),
            pl.BlockSpec((HALF, HID_DIM), lambda i: (0, 0)),
            pl.BlockSpec((HALF, HID_DIM), lambda i: (0, 0)),
            pl.BlockSpec((1, HID_DIM), lambda i: (0, 0)),
        ],
        out_specs=pl.BlockSpec((R, HID_DIM), lambda i: (i, 0)),
        out_shape=jax.ShapeDtypeStruct((N_NODES, HID_DIM), jnp.float32),
    )(x, neigh, ws, wn0, wn1, b)


@jax.jit
def kernel(item_emb, edges, w_self_W, w_self_b, w_neigh_W, w_neigh_b):
    f32 = jnp.float32
    src = edges[:, 0].astype(jnp.int32)
    dst = edges[:, 1].astype(jnp.int32)
    ones = jnp.ones((N_NODES, 1), f32)
    pad = jnp.zeros((N_NODES, WCOLS - HALF - 1), f32)
    aug = jnp.concatenate([
        jnp.concatenate([item_emb[:, :HALF], ones, pad], axis=1),
        jnp.concatenate([item_emb[:, HALF:], ones, pad], axis=1),
    ], axis=0)
    src3 = src.reshape(NS, NCHUNK, CHUNK)
    dst4 = jnp.stack([dst, dst + N_NODES]).reshape(NC, NS, NCHUNK, CHUNK)
    zrows = jnp.zeros((ROWS_PER_TILE, WCOLS), f32)

    neigh = _sc_scatter_call(aug, src3, dst4, zrows)

    bias = (w_self_b + w_neigh_b).reshape(1, HID_DIM)
    return _tc_call(item_emb, neigh, w_self_W,
                    w_neigh_W[:HALF], w_neigh_W[HALF:], bias)
